# baseline (speedup 1.0000x reference)
import jax
import jax.numpy as jnp
from jax.experimental import pallas as pl

NUM_CLASSES = 100000
BLOCK_C = 2048


def _onehot_block(users_ref, out_ref):
    j = pl.program_id(0)
    base = j * BLOCK_C
    cols = jax.lax.broadcasted_iota(jnp.int32, out_ref.shape, 1) + base
    out_ref[:, :] = (users_ref[:, :] == cols).astype(jnp.float32)


def kernel(users):
    b = users.shape[0]
    users2 = users.reshape(b, 1)
    return pl.pallas_call(
        _onehot_block,
        grid=(pl.cdiv(NUM_CLASSES, BLOCK_C),),
        in_specs=[pl.BlockSpec((b, 1), lambda j: (0, 0))],
        out_specs=pl.BlockSpec((b, BLOCK_C), lambda j: (0, j)),
        out_shape=jax.ShapeDtypeStruct((b, NUM_CLASSES), jnp.float32),
    )(users2)


# speedup vs baseline: 1.0194x; 1.0194x over previous
import jax
import jax.numpy as jnp
from jax.experimental import pallas as pl
from jax.experimental.pallas import tpu as pltpu

NUM_CLASSES = 100000
BLOCK_C = 4096


def _onehot_block(users_ref, out_ref):
    j = pl.program_id(0)
    base = j * BLOCK_C
    cols = jax.lax.broadcasted_iota(jnp.int32, out_ref.shape, 1) + base
    out_ref[:, :] = (users_ref[:, :] == cols).astype(jnp.float32)


def kernel(users):
    b = users.shape[0]
    users2 = users.reshape(b, 1)
    return pl.pallas_call(
        _onehot_block,
        grid=(pl.cdiv(NUM_CLASSES, BLOCK_C),),
        in_specs=[pl.BlockSpec((b, 1), lambda j: (0, 0))],
        out_specs=pl.BlockSpec((b, BLOCK_C), lambda j: (0, j)),
        out_shape=jax.ShapeDtypeStruct((b, NUM_CLASSES), jnp.float32),
    )(users2)
